# baseline (speedup 1.0000x reference)
import functools

import jax
import jax.numpy as jnp
from jax import lax
from jax.experimental import pallas as pl
from jax.experimental.pallas import tpu as pltpu
from jax.experimental.pallas import tpu_sc as plsc

_NC = 2
_NS = 16
_NW = _NC * _NS
_L = 16


def _sc_gather4(table4, idx):
    B, = idx.shape
    D4 = table4.shape[1]
    b_per_w = B // _NW

    @functools.partial(
        pl.kernel,
        mesh=plsc.VectorSubcoreMesh(core_axis_name="c", subcore_axis_name="s"),
        out_type=jax.ShapeDtypeStruct((B, D4), table4.dtype),
        scratch_types=[
            pltpu.VMEM((b_per_w,), jnp.int32),
            pltpu.VMEM((b_per_w,), jnp.int32),
            pltpu.VMEM((b_per_w, D4), table4.dtype),
            pltpu.SemaphoreType.DMA,
        ],
    )
    def gather_k(table_hbm, idx_hbm, out_hbm, idx_v, idx2_v, rows_v, sem):
        wid = lax.axis_index("s") * _NC + lax.axis_index("c")
        base = wid * b_per_w
        pltpu.sync_copy(idx_hbm.at[pl.ds(base, b_per_w)], idx_v)
        for g in range(b_per_w // _L):
            v = idx_v[pl.ds(g * _L, _L)]
            idx2_v[pl.ds(g * _L, _L)] = lax.shift_right_logical(v, 2)
        pltpu.async_copy(table_hbm.at[idx2_v], rows_v, sem).wait()
        pltpu.sync_copy(rows_v, out_hbm.at[pl.ds(base, b_per_w)])

    return gather_k(table4, idx)


def _tc_score_body(theta4_ref, uidx_ref, alpha_ref, out_ref):
    sub = uidx_ref[...] & 3
    t4 = theta4_ref[...]
    D = t4.shape[1] // 4
    theta = jnp.where(sub == 0, t4[:, 0:D], t4[:, D:2 * D])
    theta = jnp.where(sub == 2, t4[:, 2 * D:3 * D], theta)
    theta = jnp.where(sub == 3, t4[:, 3 * D:4 * D], theta)
    u = jnp.dot(theta, alpha_ref[...], preferred_element_type=jnp.float32)
    m = jnp.max(u, axis=-1, keepdims=True)
    s = u - m
    lse = jnp.log(jnp.sum(jnp.exp(s), axis=-1, keepdims=True))
    out_ref[...] = s - lse


def _tc_score(theta4, uidx, alpha_t, block_b=512):
    B, D4 = theta4.shape
    N = alpha_t.shape[1]
    return pl.pallas_call(
        _tc_score_body,
        grid=(B // block_b,),
        in_specs=[
            pl.BlockSpec((block_b, D4), lambda i: (i, 0)),
            pl.BlockSpec((block_b, 1), lambda i: (i, 0)),
            pl.BlockSpec((alpha_t.shape[0], N), lambda i: (0, 0)),
        ],
        out_specs=pl.BlockSpec((block_b, N), lambda i: (i, 0)),
        out_shape=jax.ShapeDtypeStruct((B, N), jnp.float32),
    )(theta4, uidx, alpha_t)


def kernel(user_index, theta_mean, alpha_mean):
    V, D = theta_mean.shape
    table4 = theta_mean.reshape(V // 4, 4 * D)
    idx = user_index.astype(jnp.int32)
    theta4 = _sc_gather4(table4, idx)
    alpha_t = alpha_mean.T
    return _tc_score(theta4, idx.reshape(-1, 1), alpha_t)


# speedup vs baseline: 1.0005x; 1.0005x over previous
import functools

import jax
import jax.numpy as jnp
from jax import lax
from jax.experimental import pallas as pl
from jax.experimental.pallas import tpu as pltpu
from jax.experimental.pallas import tpu_sc as plsc

_NC = 2
_NS = 16
_NW = _NC * _NS
_L = 16


def _sc_gather4(table4, idx):
    B, = idx.shape
    D4 = table4.shape[1]
    b_per_w = B // _NW

    @functools.partial(
        pl.kernel,
        mesh=plsc.VectorSubcoreMesh(core_axis_name="c", subcore_axis_name="s"),
        out_type=jax.ShapeDtypeStruct((B, D4), table4.dtype),
        scratch_types=[
            pltpu.VMEM((b_per_w,), jnp.int32),
            pltpu.VMEM((b_per_w,), jnp.int32),
            pltpu.VMEM((b_per_w, D4), table4.dtype),
            pltpu.SemaphoreType.DMA,
        ],
        compiler_params=pltpu.CompilerParams(use_tc_tiling_on_sc=True),
    )
    def gather_k(table_hbm, idx_hbm, out_hbm, idx_v, idx2_v, rows_v, sem):
        wid = lax.axis_index("s") * _NC + lax.axis_index("c")
        base = wid * b_per_w
        pltpu.sync_copy(idx_hbm.at[pl.ds(base, b_per_w)], idx_v)
        for g in range(b_per_w // _L):
            v = idx_v[pl.ds(g * _L, _L)]
            idx2_v[pl.ds(g * _L, _L)] = lax.shift_right_logical(v, 2)
        pltpu.async_copy(table_hbm.at[idx2_v], rows_v, sem).wait()
        pltpu.sync_copy(rows_v, out_hbm.at[pl.ds(base, b_per_w)])

    return gather_k(table4, idx)


def _tc_score_body(theta4_ref, uidx_ref, alpha_ref, out_ref):
    sub = uidx_ref[...] & 3
    t4 = theta4_ref[...]
    D = t4.shape[1] // 4
    theta = jnp.where(sub == 0, t4[:, 0:D], t4[:, D:2 * D])
    theta = jnp.where(sub == 2, t4[:, 2 * D:3 * D], theta)
    theta = jnp.where(sub == 3, t4[:, 3 * D:4 * D], theta)
    u = jnp.dot(theta, alpha_ref[...], preferred_element_type=jnp.float32)
    m = jnp.max(u, axis=-1, keepdims=True)
    s = u - m
    lse = jnp.log(jnp.sum(jnp.exp(s), axis=-1, keepdims=True))
    out_ref[...] = s - lse


def _tc_score(theta4, uidx, alpha_t, block_b=512):
    B, D4 = theta4.shape
    N = alpha_t.shape[1]
    return pl.pallas_call(
        _tc_score_body,
        grid=(B // block_b,),
        in_specs=[
            pl.BlockSpec((block_b, D4), lambda i: (i, 0)),
            pl.BlockSpec((block_b, 1), lambda i: (i, 0)),
            pl.BlockSpec((alpha_t.shape[0], N), lambda i: (0, 0)),
        ],
        out_specs=pl.BlockSpec((block_b, N), lambda i: (i, 0)),
        out_shape=jax.ShapeDtypeStruct((B, N), jnp.float32),
    )(theta4, uidx, alpha_t)


def kernel(user_index, theta_mean, alpha_mean):
    V, D = theta_mean.shape
    table4 = theta_mean.reshape(V // 4, 4 * D)
    idx = user_index.astype(jnp.int32)
    theta4 = _sc_gather4(table4, idx)
    alpha_t = alpha_mean.T
    return _tc_score(theta4, idx.reshape(-1, 1), alpha_t)
